# baseline (speedup 1.0000x reference)
import jax
import jax.numpy as jnp
from jax import lax
from jax.experimental import pallas as pl
from jax.experimental.pallas import tpu as pltpu
from jax.experimental.pallas import tpu_sc as plsc

B = 16384
NCAT = 20
NNUM = 6
NCOL = NCAT + NNUM
VCAT = 100000
VNUM = 1000
D = 32
PK = 4

NC = 2
NS = 16
NW = NC * NS
BPW = B // NW
NB = 128
NBLK = BPW // NB
NGRP = NB // 16
CHUNK = NCOL * NB
DU = 2
S_COL = (D // 8) * (B // NB) * 8 * NB
S_R = (B // NB) * 8 * NB
S_BLK = 8 * NB


def _sc_body(cat_pk, num_pk, idx_flat, bias_rep, out,
             idx_v, idxd0, idxd1, rows0, rows1, blk0, blk1, bias_v,
             sem_g0, sem_g1, sem_o0, sem_o1):
    wid = lax.axis_index("s") * NC + lax.axis_index("c")
    pltpu.sync_copy(bias_rep, bias_v)
    iota = lax.iota(jnp.int32, 16)
    row_vecs = [iota + g * 16 for g in range(NGRP)]

    def fire_gather(c, cat, par0):
        tab = cat_pk if cat else num_pk
        idxd = idxd0 if par0 else idxd1
        rv = rows0 if par0 else rows1
        sem = sem_g0 if par0 else sem_g1
        for g in range(NGRP):
            iv = idx_v[pl.ds(c * NB + g * 16, 16)]
            idxd[pl.ds(g * 16, 16)] = lax.shift_right_logical(iv, 2)
        return pltpu.async_copy(tab.at[idxd], rv, sem)

    def drain_out(par, n=4):
        bv = blk0 if par == 0 else blk1
        sem = sem_o0 if par == 0 else sem_o1
        for _ in range(n):
            pltpu.make_async_copy(bv.at[pl.ds(0, S_BLK)],
                                  out.at[pl.ds(0, S_BLK)], sem).wait()

    def wait_gather(par):
        rv = rows0 if par == 0 else rows1
        sem = sem_g0 if par == 0 else sem_g1
        pltpu.make_async_copy(cat_pk.at[pl.ds(0, NB)], rv, sem).wait()

    def process(c, g_id, first):
        rv = rows0 if first else rows1
        bv = blk0 if first else blk1
        offs = []
        for g in range(NGRP):
            iv = idx_v[pl.ds(c * NB + g * 16, 16)]
            offs.append(lax.shift_left(lax.bitwise_and(iv, 3), 5))

        def dim_pass(dv, carry):
            for u in range(DU):
                d = dv * DU + u
                bias_vec = bias_v[pl.ds((c * D) * 16 + d * 16, 16)]
                for g in range(NGRP):
                    v = plsc.load_gather(rv, [row_vecs[g], offs[g] + d])
                    bv[pl.ds(d * NB + g * 16, 16)] = v + bias_vec
            return carry

        lax.fori_loop(0, D // DU, dim_pass, None)
        sem_o = sem_o0 if first else sem_o1
        base = c * S_COL + g_id * S_BLK
        for r in range(D // 8):
            pltpu.async_copy(bv.at[pl.ds(r * S_BLK, S_BLK)],
                             out.at[pl.ds(base + r * S_R, S_BLK)], sem_o)

    def do_block(blk, _):
        g_id = wid * NBLK + blk
        pltpu.sync_copy(idx_flat.at[pl.ds(g_id * CHUNK, CHUNK)], idx_v)

        @pl.when(blk > 0)
        def _():
            drain_out(0)
            drain_out(1)

        fire_gather(0, True, True)

        def cat_pair(j, carry):
            fire_gather(2 * j + 1, True, False)
            wait_gather(0)

            @pl.when(j > 0)
            def _():
                drain_out(0)
                drain_out(1)

            process(2 * j, g_id, True)

            @pl.when(j < NCAT // 2 - 1)
            def _():
                fire_gather(2 * j + 2, True, True)

            wait_gather(1)
            process(2 * j + 1, g_id, False)
            return carry

        lax.fori_loop(0, NCAT // 2, cat_pair, None)

        pending = fire_gather(NCAT, False, True)
        for c in range(NCAT, NCOL):
            nxt = (fire_gather(c + 1, False, c % 2 == 1)
                   if c + 1 < NCOL else None)
            pending.wait()
            pending = nxt
            drain_out(c % 2)
            process(c, g_id, c % 2 == 0)
        return _

    lax.fori_loop(0, NBLK, do_block, None)
    drain_out(0)
    drain_out(1)


@jax.jit
def kernel(cat_idx, num_idx, cat_tables, cat_bias, num_tables, num_bias):
    idx_cat = cat_idx + jnp.arange(NCAT, dtype=jnp.int32)[None, :] * VCAT
    idx_num = num_idx + jnp.arange(NNUM, dtype=jnp.int32)[None, :] * VNUM
    idx_all = jnp.concatenate([idx_cat, idx_num], axis=1)
    idx_flat = idx_all.reshape(B // NB, NB, NCOL).transpose(0, 2, 1).reshape(-1)

    cat_pk = cat_tables.reshape(NCAT * VCAT // PK, PK * D)
    num_pk = jnp.pad(num_tables.reshape(NNUM * VNUM, D),
                     ((0, 16), (0, 0))).reshape((NNUM * VNUM + 16) // PK,
                                                PK * D)
    bias_rep = jnp.repeat(
        jnp.concatenate([cat_bias, num_bias], axis=0).reshape(-1), 16)

    mesh = plsc.VectorSubcoreMesh(core_axis_name="c", subcore_axis_name="s")
    out = pl.kernel(
        _sc_body,
        mesh=mesh,
        compiler_params=pltpu.CompilerParams(use_tc_tiling_on_sc=False,
                                             needs_layout_passes=False),
        out_type=jax.ShapeDtypeStruct((B * NCOL * D,), jnp.float32),
        scratch_types=[
            pltpu.VMEM((CHUNK,), jnp.int32),
            pltpu.VMEM((NB,), jnp.int32),
            pltpu.VMEM((NB,), jnp.int32),
            pltpu.VMEM((NB, PK * D), jnp.float32),
            pltpu.VMEM((NB, PK * D), jnp.float32),
            pltpu.VMEM((D * NB,), jnp.float32),
            pltpu.VMEM((D * NB,), jnp.float32),
            pltpu.VMEM((NCOL * D * 16,), jnp.float32),
            pltpu.SemaphoreType.DMA,
            pltpu.SemaphoreType.DMA,
            pltpu.SemaphoreType.DMA,
            pltpu.SemaphoreType.DMA,
        ],
    )(cat_pk, num_pk, idx_flat, bias_rep)

    x = out.reshape(NCOL, D // 8, B // NB, 8, NB)
    x = x.transpose(0, 1, 3, 2, 4)
    x = x.reshape(NCOL, D, B)
    return x.transpose(2, 0, 1)
